# baseline (speedup 1.0000x reference)
import functools

import jax
import jax.numpy as jnp
from jax import lax
from jax.experimental import pallas as pl
from jax.experimental.pallas import tpu as pltpu
from jax.experimental.pallas import tpu_sc as plsc


def _lrelu(v):
    return jnp.where(v >= 0, v, 0.2 * v)


def _prep_body(x_ref, wa_ref, w2_ref, b2_ref, ba_ref, t_ref, wc_ref, bc_ref):
    oh = w2_ref.shape[0]
    wa = wa_ref[...]
    wa_m = wa[:oh, :]
    wa_x = wa[oh:, :]
    t_ref[...] = jnp.dot(x_ref[...], wa_x, preferred_element_type=jnp.float32)
    wc_ref[...] = jnp.dot(w2_ref[...], wa_m, preferred_element_type=jnp.float32)
    bc_ref[...] = (
        jnp.dot(b2_ref[...], wa_m, preferred_element_type=jnp.float32) + ba_ref[...]
    )


def _pass1_body(
    xs_ref, ea_ref, tt_ref, w1_ref, b1_ref, wc_ref, bc_ref,
    out_ref, m_ref, z_ref, macc, sacc,
):
    nin = xs_ref.shape[1]
    w1 = w1_ref[...]
    xb = xs_ref[...].astype(jnp.bfloat16)
    eb = ea_ref[...]
    pre = (
        jnp.dot(xb, w1[:nin, :], preferred_element_type=jnp.float32)
        + jnp.dot(eb, w1[nin:, :], preferred_element_type=jnp.float32)
        + b1_ref[...]
    )
    h = _lrelu(pre).astype(jnp.bfloat16)
    l = (
        jnp.dot(h, wc_ref[...], preferred_element_type=jnp.float32)
        + bc_ref[...]
        + tt_ref[...]
    )
    out_ref[...] = l

    i = pl.program_id(0)
    bm = jnp.max(l, axis=0, keepdims=True)

    @pl.when(i == 0)
    def _init():
        macc[...] = bm
        sacc[...] = jnp.sum(jnp.exp(l - bm), axis=0, keepdims=True)

    @pl.when(i > 0)
    def _update():
        mo = macc[...]
        mn = jnp.maximum(mo, bm)
        sacc[...] = sacc[...] * jnp.exp(mo - mn) + jnp.sum(
            jnp.exp(l - mn), axis=0, keepdims=True
        )
        macc[...] = mn

    m_ref[...] = macc[...]
    z_ref[...] = sacc[...]


def _reduce_body(l_ref, m_ref, z_ref, macc, sacc):
    i = pl.program_id(0)
    l = l_ref[...]
    bm = jnp.max(l, axis=0, keepdims=True)

    @pl.when(i == 0)
    def _init():
        macc[...] = bm
        sacc[...] = jnp.sum(jnp.exp(l - bm), axis=0, keepdims=True)

    @pl.when(i > 0)
    def _update():
        mo = macc[...]
        mn = jnp.maximum(mo, bm)
        sacc[...] = sacc[...] * jnp.exp(mo - mn) + jnp.sum(
            jnp.exp(l - mn), axis=0, keepdims=True
        )
        macc[...] = mn

    m_ref[...] = macc[...]
    z_ref[...] = sacc[...]


def _pass2_body(
    xs_ref, ea_ref, l_ref, m0_ref, z0_ref, m1_ref, z1_ref, w1_ref, b1_ref,
    w2_ref, sel_ref, out_ref,
):
    nin = xs_ref.shape[1]
    out = out_ref.shape[1]
    w1 = w1_ref[...]
    xb = xs_ref[...].astype(jnp.bfloat16)
    eb = ea_ref[...]
    pre = (
        jnp.dot(xb, w1[:nin, :], preferred_element_type=jnp.float32)
        + jnp.dot(eb, w1[nin:, :], preferred_element_type=jnp.float32)
        + b1_ref[...]
    )
    h = _lrelu(pre).astype(jnp.bfloat16)
    mo0 = m0_ref[...]
    mo1 = m1_ref[...]
    mg = jnp.maximum(mo0, mo1)
    zg = z0_ref[...] * jnp.exp(mo0 - mg) + z1_ref[...] * jnp.exp(mo1 - mg)
    w = jnp.exp(l_ref[...] - mg) / zg * 0.25
    w2 = w2_ref[...]
    acc = jnp.dot(w, sel_ref[...], preferred_element_type=jnp.float32)
    for hd in range(4):
        mh = jnp.dot(
            h, w2[:, hd * out : (hd + 1) * out], preferred_element_type=jnp.float32
        )
        acc = acc + mh * w[:, hd : hd + 1]
    out_ref[...] = acc


def _pass3_body(
    x_ref, a0_ref, a1_ref, a2_ref, a3_ref, wu_ref, bu_ref, g_ref, b_ref, out_ref
):
    nin = x_ref.shape[1]
    wu = wu_ref[...]
    ag = (a0_ref[...] + a1_ref[...]) + (a2_ref[...] + a3_ref[...])
    u = (
        jnp.dot(x_ref[...], wu[:nin, :], precision=lax.Precision.HIGHEST,
                preferred_element_type=jnp.float32)
        + jnp.dot(ag, wu[nin:, :], precision=lax.Precision.HIGHEST,
                  preferred_element_type=jnp.float32)
        + bu_ref[...]
    )
    mean = jnp.mean(u, axis=-1, keepdims=True)
    cen = u - mean
    var = jnp.mean(cen * cen, axis=-1, keepdims=True)
    un = cen * lax.rsqrt(var + 1e-5) * g_ref[...] + b_ref[...]
    out_ref[...] = _lrelu(un)


_CH = 128
_GCH = 64


def _sc_gather_body(
    x_hbm, t_hbm, src_hbm, tgt_hbm, xs_out, tt_out,
    idx_s0, idx_s1, idx_t0, idx_t1, rows0, rows1, tbuf0, tbuf1, tvm,
    sem_g, sem_i0, sem_i1, sem_ox0, sem_ox1, sem_ot0, sem_ot1,
):
    idx_s = [idx_s0, idx_s1]
    idx_t = [idx_t0, idx_t1]
    rows = [rows0, rows1]
    tbuf = [tbuf0, tbuf1]
    sem_i = [sem_i0, sem_i1]
    sem_ox = [sem_ox0, sem_ox1]
    sem_ot = [sem_ot0, sem_ot1]
    e = src_hbm.shape[0]
    nch = e // _GCH
    nw = 32
    wid = lax.axis_index("s") * 2 + lax.axis_index("c")
    base_n = nch // nw
    rem = nch - base_n * nw
    nk = jnp.where(wid < rem, base_n + 1, base_n)

    pltpu.sync_copy(t_hbm, tvm)
    zero16 = jnp.zeros((16,), jnp.float32)
    for b in range(2):
        for r in range(_GCH):
            tbuf[b][r, :] = zero16
    lane = lax.iota(jnp.int32, 16)

    def chunk_off(i):
        return pl.multiple_of((wid + i * nw) * _GCH, _GCH)

    def issue_idx(i, b):
        off = chunk_off(i)
        pltpu.async_copy(src_hbm.at[pl.ds(off, _GCH)], idx_s[b], sem_i[b])
        pltpu.async_copy(tgt_hbm.at[pl.ds(off, _GCH)], idx_t[b], sem_i[b])

    def wait_idx(b):
        pltpu.make_async_copy(src_hbm.at[pl.ds(0, _GCH)], idx_s[b], sem_i[b]).wait()
        pltpu.make_async_copy(tgt_hbm.at[pl.ds(0, _GCH)], idx_t[b], sem_i[b]).wait()

    def wait_out(b):
        pltpu.make_async_copy(rows[b], xs_out.at[pl.ds(0, _GCH)], sem_ox[b]).wait()
        pltpu.make_async_copy(tbuf[b], tt_out.at[pl.ds(0, _GCH)], sem_ot[b]).wait()

    def step(i, b, bo):
        @pl.when(i < nk)
        def _():
            @pl.when(i + 1 < nk)
            def _():
                issue_idx(i + 1, bo)

            wait_idx(b)

            @pl.when(i >= 2)
            def _():
                wait_out(b)

            g = pltpu.async_copy(x_hbm.at[idx_s[b]], rows[b], sem_g)
            for v in range(_GCH // 16):
                tv = idx_t[b][pl.ds(v * 16, 16)]
                row_i = lane + v * 16
                for j in range(4):
                    vals = plsc.load_gather(tvm, [tv * 4 + j])
                    plsc.store_scatter(
                        tbuf[b], [row_i, jnp.full((16,), j, jnp.int32)], vals
                    )
            g.wait()
            off = chunk_off(i)
            pltpu.async_copy(rows[b], xs_out.at[pl.ds(off, _GCH)], sem_ox[b])
            pltpu.async_copy(tbuf[b], tt_out.at[pl.ds(off, _GCH)], sem_ot[b])

    issue_idx(0, 0)

    def pair(j, carry):
        step(2 * j, 0, 1)
        step(2 * j + 1, 1, 0)
        return carry

    lax.fori_loop(0, (base_n + 2) // 2, pair, 0)
    wait_out(0)
    wait_out(1)


def _sc_scatter_body(
    w_hbm, tgt_hbm, z_hbm, out0, out1,
    idx0, idx1, rows0, rows1,
    sem_i0, sem_i1, sem_r0, sem_r1, sem_s0, sem_s1,
):
    idx = [idx0, idx1]
    rows = [rows0, rows1]
    sem_i = [sem_i0, sem_i1]
    sem_r = [sem_r0, sem_r1]
    sem_s = [sem_s0, sem_s1]
    e = tgt_hbm.shape[0]
    nch = e // _CH
    n0 = (nch + 1) // 2
    c = lax.axis_index("c")
    s = lax.axis_index("s")

    pltpu.sync_copy(z_hbm, rows0)

    def _zero_plane(plane):
        for j in range(5):
            pltpu.sync_copy(rows0, plane.at[pl.ds(s * 640 + j * _CH, _CH)])

    @pl.when(c == 0)
    def _z0():
        _zero_plane(out0)

    @pl.when(c == 1)
    def _z1():
        _zero_plane(out1)

    plsc.subcore_barrier()

    cnt = jnp.where(c == 0, n0, nch - n0)
    base_n = cnt // 16
    rem = cnt - base_n * 16
    nk = jnp.where(s < rem, base_n + 1, base_n)

    def chunk_off(i):
        return pl.multiple_of((c * n0 + s + i * 16) * _CH, _CH)

    def issue_stage(i, b):
        off = chunk_off(i)
        pltpu.async_copy(tgt_hbm.at[pl.ds(off, _CH)], idx[b], sem_i[b])
        pltpu.async_copy(w_hbm.at[pl.ds(off, _CH)], rows[b], sem_r[b])

    def wait_stage(b):
        pltpu.make_async_copy(tgt_hbm.at[pl.ds(0, _CH)], idx[b], sem_i[b]).wait()
        pltpu.make_async_copy(w_hbm.at[pl.ds(0, _CH)], rows[b], sem_r[b]).wait()

    def wait_scat(b):
        @pl.when(c == 0)
        def _():
            pltpu.make_async_copy(rows[b], out0.at[pl.ds(0, _CH)], sem_s[b]).wait()

        @pl.when(c == 1)
        def _():
            pltpu.make_async_copy(rows[b], out1.at[pl.ds(0, _CH)], sem_s[b]).wait()

    def step(i, b, bo):
        @pl.when(i < nk)
        def _():
            @pl.when(i + 1 < nk)
            def _():
                @pl.when(i >= 1)
                def _():
                    wait_scat(bo)

                issue_stage(i + 1, bo)

            wait_stage(b)

            @pl.when(c == 0)
            def _():
                pltpu.async_copy(rows[b], out0.at[idx[b]], sem_s[b], add=True)

            @pl.when(c == 1)
            def _():
                pltpu.async_copy(rows[b], out1.at[idx[b]], sem_s[b], add=True)

    issue_stage(0, 0)

    def pair(j, carry):
        step(2 * j, 0, 1)
        step(2 * j + 1, 1, 0)
        return carry

    lax.fori_loop(0, (n0 // 16 + 2) // 2, pair, 0)
    wait_scat(0)
    wait_scat(1)


def kernel(x, edge_index, edge_attr, W1, b1, W2, b2, Wa, ba, Wu, bu, gamma, beta):
    n, nin = x.shape
    e = edge_index.shape[1]
    ed = edge_attr.shape[1]
    oh = W2.shape[0]
    nh = Wa.shape[1]
    out = Wu.shape[1]
    f32 = jnp.float32

    src = edge_index[0]
    tgt = edge_index[1]

    b1r = b1.reshape(1, oh)
    b2r = b2.reshape(1, oh)
    bur = bu.reshape(1, out)
    gr = gamma.reshape(1, out)
    br = beta.reshape(1, out)
    wa16 = jnp.pad(Wa, ((0, 0), (0, 16 - nh)))
    ba16 = jnp.pad(ba.reshape(1, nh), ((0, 0), (0, 16 - nh)))
    w1b = W1.astype(jnp.bfloat16)
    w2b = W2.astype(jnp.bfloat16)
    eab16 = edge_attr.astype(jnp.bfloat16)
    sel = jnp.pad(b2.reshape(nh, out), ((0, 16 - nh), (0, 0)))

    t16, wc16, bc16 = pl.pallas_call(
        _prep_body,
        out_shape=[
            jax.ShapeDtypeStruct((n, 16), f32),
            jax.ShapeDtypeStruct((oh, 16), f32),
            jax.ShapeDtypeStruct((1, 16), f32),
        ],
    )(x, wa16, W2, b2r, ba16)
    wc16b = wc16.astype(jnp.bfloat16)
    t4 = t16[:, :nh].reshape(-1)

    mesh = plsc.VectorSubcoreMesh(core_axis_name="c", subcore_axis_name="s")
    gather_k = functools.partial(
        pl.kernel,
        out_type=[
            jax.ShapeDtypeStruct((e // 2, nin), f32),
            jax.ShapeDtypeStruct((e // 2, 16), f32),
        ],
        mesh=mesh,
        compiler_params=pltpu.CompilerParams(needs_layout_passes=False),
        scratch_types=[
            pltpu.VMEM((_GCH,), jnp.int32),
            pltpu.VMEM((_GCH,), jnp.int32),
            pltpu.VMEM((_GCH,), jnp.int32),
            pltpu.VMEM((_GCH,), jnp.int32),
            pltpu.VMEM((_GCH, nin), f32),
            pltpu.VMEM((_GCH, nin), f32),
            pltpu.VMEM((_GCH, 16), f32),
            pltpu.VMEM((_GCH, 16), f32),
            pltpu.VMEM((nh * n,), f32),
            pltpu.SemaphoreType.DMA,
            pltpu.SemaphoreType.DMA,
            pltpu.SemaphoreType.DMA,
            pltpu.SemaphoreType.DMA,
            pltpu.SemaphoreType.DMA,
            pltpu.SemaphoreType.DMA,
            pltpu.SemaphoreType.DMA,
        ],
    )(_sc_gather_body)

    eh = e // 2
    be = 3200
    be2 = 2000

    def run_gather(sl):
        return gather_k(x, t4, src[sl], tgt[sl])

    def run_pass1(xsrc_h, ea_h, ttgt_h):
        return pl.pallas_call(
            _pass1_body,
            grid=(eh // be,),
            in_specs=[
                pl.BlockSpec((be, nin), lambda i: (i, 0)),
                pl.BlockSpec((be, ed), lambda i: (i, 0)),
                pl.BlockSpec((be, 16), lambda i: (i, 0)),
                pl.BlockSpec((nin + ed, oh), lambda i: (0, 0)),
                pl.BlockSpec((1, oh), lambda i: (0, 0)),
                pl.BlockSpec((oh, 16), lambda i: (0, 0)),
                pl.BlockSpec((1, 16), lambda i: (0, 0)),
            ],
            out_specs=[
                pl.BlockSpec((be, 16), lambda i: (i, 0)),
                pl.BlockSpec((1, 16), lambda i: (0, 0)),
                pl.BlockSpec((1, 16), lambda i: (0, 0)),
            ],
            out_shape=[
                jax.ShapeDtypeStruct((eh, 16), f32),
                jax.ShapeDtypeStruct((1, 16), f32),
                jax.ShapeDtypeStruct((1, 16), f32),
            ],
            scratch_shapes=[
                pltpu.VMEM((1, 16), f32),
                pltpu.VMEM((1, 16), f32),
            ],
        )(xsrc_h, ea_h, ttgt_h, w1b, b1r, wc16b, bc16)

    def run_pass2(xsrc_h, ea_h, logits_h, m0, z0, m1, z1):
        return pl.pallas_call(
            _pass2_body,
            grid=(eh // be2,),
            in_specs=[
                pl.BlockSpec((be2, nin), lambda i: (i, 0)),
                pl.BlockSpec((be2, ed), lambda i: (i, 0)),
                pl.BlockSpec((be2, 16), lambda i: (i, 0)),
                pl.BlockSpec((1, 16), lambda i: (0, 0)),
                pl.BlockSpec((1, 16), lambda i: (0, 0)),
                pl.BlockSpec((1, 16), lambda i: (0, 0)),
                pl.BlockSpec((1, 16), lambda i: (0, 0)),
                pl.BlockSpec((nin + ed, oh), lambda i: (0, 0)),
                pl.BlockSpec((1, oh), lambda i: (0, 0)),
                pl.BlockSpec((oh, oh), lambda i: (0, 0)),
                pl.BlockSpec((16, out), lambda i: (0, 0)),
            ],
            out_specs=pl.BlockSpec((be2, out), lambda i: (i, 0)),
            out_shape=jax.ShapeDtypeStruct((eh, out), f32),
        )(xsrc_h, ea_h, logits_h, m0, z0, m1, z1, w1b, b1r, w2b, sel)

    zeros_hbm = jnp.zeros((_CH, out), f32)
    scatter_k = functools.partial(
        pl.kernel,
        out_type=[
            jax.ShapeDtypeStruct((10240, out), f32),
            jax.ShapeDtypeStruct((10240, out), f32),
        ],
        mesh=mesh,
        compiler_params=pltpu.CompilerParams(needs_layout_passes=False),
        scratch_types=[
            pltpu.VMEM((_CH,), jnp.int32),
            pltpu.VMEM((_CH,), jnp.int32),
            pltpu.VMEM((_CH, out), f32),
            pltpu.VMEM((_CH, out), f32),
            pltpu.SemaphoreType.DMA,
            pltpu.SemaphoreType.DMA,
            pltpu.SemaphoreType.DMA,
            pltpu.SemaphoreType.DMA,
            pltpu.SemaphoreType.DMA,
            pltpu.SemaphoreType.DMA,
        ],
    )(_sc_scatter_body)

    h0 = slice(0, eh)
    h1 = slice(eh, e)
    ea0, ea1 = eab16[h0], eab16[h1]
    xsrc0, ttgt0 = run_gather(h0)
    xsrc1, ttgt1 = run_gather(h1)
    logits0, m0, z0 = run_pass1(xsrc0, ea0, ttgt0)
    logits1, m1, z1 = run_pass1(xsrc1, ea1, ttgt1)
    weighted0 = run_pass2(xsrc0, ea0, logits0, m0, z0, m1, z1)
    agg0a, agg1a = scatter_k(weighted0, tgt[h0], zeros_hbm)
    weighted1 = run_pass2(xsrc1, ea1, logits1, m0, z0, m1, z1)
    agg0b, agg1b = scatter_k(weighted1, tgt[h1], zeros_hbm)

    bn = 2000
    return pl.pallas_call(
        _pass3_body,
        grid=(n // bn,),
        in_specs=[
            pl.BlockSpec((bn, nin), lambda i: (i, 0)),
            pl.BlockSpec((bn, out), lambda i: (i, 0)),
            pl.BlockSpec((bn, out), lambda i: (i, 0)),
            pl.BlockSpec((bn, out), lambda i: (i, 0)),
            pl.BlockSpec((bn, out), lambda i: (i, 0)),
            pl.BlockSpec((nin + out, out), lambda i: (0, 0)),
            pl.BlockSpec((1, out), lambda i: (0, 0)),
            pl.BlockSpec((1, out), lambda i: (0, 0)),
            pl.BlockSpec((1, out), lambda i: (0, 0)),
        ],
        out_specs=pl.BlockSpec((bn, out), lambda i: (i, 0)),
        out_shape=jax.ShapeDtypeStruct((n, out), f32),
    )(x, agg0a[:n], agg1a[:n], agg0b[:n], agg1b[:n], Wu, bur, gr, br)


# speedup vs baseline: 1.0028x; 1.0028x over previous
import functools

import jax
import jax.numpy as jnp
from jax import lax
from jax.experimental import pallas as pl
from jax.experimental.pallas import tpu as pltpu
from jax.experimental.pallas import tpu_sc as plsc


def _lrelu(v):
    return jnp.where(v >= 0, v, 0.2 * v)


def _prep_body(x_ref, wa_ref, w2_ref, b2_ref, ba_ref, t_ref, wc_ref, bc_ref):
    oh = w2_ref.shape[0]
    wa = wa_ref[...]
    wa_m = wa[:oh, :]
    wa_x = wa[oh:, :]
    t_ref[...] = jnp.dot(x_ref[...], wa_x, preferred_element_type=jnp.float32)
    wc_ref[...] = jnp.dot(w2_ref[...], wa_m, preferred_element_type=jnp.float32)
    bc_ref[...] = (
        jnp.dot(b2_ref[...], wa_m, preferred_element_type=jnp.float32) + ba_ref[...]
    )


def _pass1_body(
    xs_ref, ea_ref, tt_ref, w1_ref, b1_ref, wc_ref, bc_ref,
    out_ref, m_ref, z_ref, macc, sacc,
):
    nin = xs_ref.shape[1]
    w1 = w1_ref[...]
    xb = xs_ref[...].astype(jnp.bfloat16)
    eb = ea_ref[...]
    pre = (
        jnp.dot(xb, w1[:nin, :], preferred_element_type=jnp.float32)
        + jnp.dot(eb, w1[nin:, :], preferred_element_type=jnp.float32)
        + b1_ref[...]
    )
    h = _lrelu(pre).astype(jnp.bfloat16)
    l = (
        jnp.dot(h, wc_ref[...], preferred_element_type=jnp.float32)
        + bc_ref[...]
        + tt_ref[...]
    )
    out_ref[...] = l

    i = pl.program_id(0)
    bm = jnp.max(l, axis=0, keepdims=True)

    @pl.when(i == 0)
    def _init():
        macc[...] = bm
        sacc[...] = jnp.sum(jnp.exp(l - bm), axis=0, keepdims=True)

    @pl.when(i > 0)
    def _update():
        mo = macc[...]
        mn = jnp.maximum(mo, bm)
        sacc[...] = sacc[...] * jnp.exp(mo - mn) + jnp.sum(
            jnp.exp(l - mn), axis=0, keepdims=True
        )
        macc[...] = mn

    m_ref[...] = macc[...]
    z_ref[...] = sacc[...]


def _pass2_body(
    xs_ref, ea_ref, l_ref, m0_ref, z0_ref, m1_ref, z1_ref, w1_ref, b1_ref,
    w2_ref, sel_ref, out_ref,
):
    nin = xs_ref.shape[1]
    out = out_ref.shape[1]
    w1 = w1_ref[...]
    xb = xs_ref[...].astype(jnp.bfloat16)
    eb = ea_ref[...]
    pre = (
        jnp.dot(xb, w1[:nin, :], preferred_element_type=jnp.float32)
        + jnp.dot(eb, w1[nin:, :], preferred_element_type=jnp.float32)
        + b1_ref[...]
    )
    h = _lrelu(pre).astype(jnp.bfloat16)
    mo0 = m0_ref[...]
    mo1 = m1_ref[...]
    mg = jnp.maximum(mo0, mo1)
    zg = z0_ref[...] * jnp.exp(mo0 - mg) + z1_ref[...] * jnp.exp(mo1 - mg)
    w = jnp.exp(l_ref[...] - mg) / zg * 0.25
    w2 = w2_ref[...]
    acc = jnp.dot(w, sel_ref[...], preferred_element_type=jnp.float32)
    for hd in range(4):
        mh = jnp.dot(
            h, w2[:, hd * out : (hd + 1) * out], preferred_element_type=jnp.float32
        )
        acc = acc + mh * w[:, hd : hd + 1]
    out_ref[...] = acc


def _pass3_body(
    x_ref, a0_ref, a1_ref, a2_ref, a3_ref, wu_ref, bu_ref, g_ref, b_ref, out_ref
):
    nin = x_ref.shape[1]
    wu = wu_ref[...]
    ag = (a0_ref[...] + a1_ref[...]) + (a2_ref[...] + a3_ref[...])
    u = (
        jnp.dot(x_ref[...], wu[:nin, :], precision=lax.Precision.HIGHEST,
                preferred_element_type=jnp.float32)
        + jnp.dot(ag, wu[nin:, :], precision=lax.Precision.HIGHEST,
                  preferred_element_type=jnp.float32)
        + bu_ref[...]
    )
    mean = jnp.mean(u, axis=-1, keepdims=True)
    cen = u - mean
    var = jnp.mean(cen * cen, axis=-1, keepdims=True)
    un = cen * lax.rsqrt(var + 1e-5) * g_ref[...] + b_ref[...]
    out_ref[...] = _lrelu(un)


_CH = 128
_GCH = 64


def _sc_gather_body(
    x_hbm, t_hbm, src_hbm, tgt_hbm, xs_out, tt_out,
    idx_s0, idx_s1, idx_t0, idx_t1, rows0, rows1, tbuf0, tbuf1, tvm,
    sem_g, sem_i0, sem_i1, sem_ox0, sem_ox1, sem_ot0, sem_ot1,
):
    idx_s = [idx_s0, idx_s1]
    idx_t = [idx_t0, idx_t1]
    rows = [rows0, rows1]
    tbuf = [tbuf0, tbuf1]
    sem_i = [sem_i0, sem_i1]
    sem_ox = [sem_ox0, sem_ox1]
    sem_ot = [sem_ot0, sem_ot1]
    e = src_hbm.shape[0]
    nch = e // _GCH
    nw = 32
    wid = lax.axis_index("s") * 2 + lax.axis_index("c")
    base_n = nch // nw
    rem = nch - base_n * nw
    nk = jnp.where(wid < rem, base_n + 1, base_n)

    pltpu.sync_copy(t_hbm, tvm)
    zero16 = jnp.zeros((16,), jnp.float32)
    for b in range(2):
        for r in range(_GCH):
            tbuf[b][r, :] = zero16
    lane = lax.iota(jnp.int32, 16)

    def chunk_off(i):
        return pl.multiple_of((wid + i * nw) * _GCH, _GCH)

    def issue_idx(i, b):
        off = chunk_off(i)
        pltpu.async_copy(src_hbm.at[pl.ds(off, _GCH)], idx_s[b], sem_i[b])
        pltpu.async_copy(tgt_hbm.at[pl.ds(off, _GCH)], idx_t[b], sem_i[b])

    def wait_idx(b):
        pltpu.make_async_copy(src_hbm.at[pl.ds(0, _GCH)], idx_s[b], sem_i[b]).wait()
        pltpu.make_async_copy(tgt_hbm.at[pl.ds(0, _GCH)], idx_t[b], sem_i[b]).wait()

    def wait_out(b):
        pltpu.make_async_copy(rows[b], xs_out.at[pl.ds(0, _GCH)], sem_ox[b]).wait()
        pltpu.make_async_copy(tbuf[b], tt_out.at[pl.ds(0, _GCH)], sem_ot[b]).wait()

    def step(i, b, bo):
        @pl.when(i < nk)
        def _():
            @pl.when(i + 1 < nk)
            def _():
                issue_idx(i + 1, bo)

            wait_idx(b)

            @pl.when(i >= 2)
            def _():
                wait_out(b)

            g = pltpu.async_copy(x_hbm.at[idx_s[b]], rows[b], sem_g)
            for v in range(_GCH // 16):
                tv = idx_t[b][pl.ds(v * 16, 16)]
                row_i = lane + v * 16
                for j in range(4):
                    vals = plsc.load_gather(tvm, [tv * 4 + j])
                    plsc.store_scatter(
                        tbuf[b], [row_i, jnp.full((16,), j, jnp.int32)], vals
                    )
            g.wait()
            off = chunk_off(i)
            pltpu.async_copy(rows[b], xs_out.at[pl.ds(off, _GCH)], sem_ox[b])
            pltpu.async_copy(tbuf[b], tt_out.at[pl.ds(off, _GCH)], sem_ot[b])

    issue_idx(0, 0)

    def pair(j, carry):
        step(2 * j, 0, 1)
        step(2 * j + 1, 1, 0)
        return carry

    lax.fori_loop(0, (base_n + 2) // 2, pair, 0)
    wait_out(0)
    wait_out(1)


def _sc_scatter_body(
    w_hbm, tgt_hbm, z_hbm, out0, out1,
    idx0, idx1, rows0, rows1,
    sem_i0, sem_i1, sem_r0, sem_r1, sem_s0, sem_s1,
):
    idx = [idx0, idx1]
    rows = [rows0, rows1]
    sem_i = [sem_i0, sem_i1]
    sem_r = [sem_r0, sem_r1]
    sem_s = [sem_s0, sem_s1]
    e = tgt_hbm.shape[0]
    nch = e // _CH
    n0 = (nch + 1) // 2
    c = lax.axis_index("c")
    s = lax.axis_index("s")

    pltpu.sync_copy(z_hbm, rows0)

    def _zero_plane(plane):
        for j in range(5):
            pltpu.sync_copy(rows0, plane.at[pl.ds(s * 640 + j * _CH, _CH)])

    @pl.when(c == 0)
    def _z0():
        _zero_plane(out0)

    @pl.when(c == 1)
    def _z1():
        _zero_plane(out1)

    plsc.subcore_barrier()

    cnt = jnp.where(c == 0, n0, nch - n0)
    base_n = cnt // 16
    rem = cnt - base_n * 16
    nk = jnp.where(s < rem, base_n + 1, base_n)

    def chunk_off(i):
        return pl.multiple_of((c * n0 + s + i * 16) * _CH, _CH)

    def issue_stage(i, b):
        off = chunk_off(i)
        pltpu.async_copy(tgt_hbm.at[pl.ds(off, _CH)], idx[b], sem_i[b])
        pltpu.async_copy(w_hbm.at[pl.ds(off, _CH)], rows[b], sem_r[b])

    def wait_stage(b):
        pltpu.make_async_copy(tgt_hbm.at[pl.ds(0, _CH)], idx[b], sem_i[b]).wait()
        pltpu.make_async_copy(w_hbm.at[pl.ds(0, _CH)], rows[b], sem_r[b]).wait()

    def wait_scat(b):
        @pl.when(c == 0)
        def _():
            pltpu.make_async_copy(rows[b], out0.at[pl.ds(0, _CH)], sem_s[b]).wait()

        @pl.when(c == 1)
        def _():
            pltpu.make_async_copy(rows[b], out1.at[pl.ds(0, _CH)], sem_s[b]).wait()

    def step(i, b, bo):
        @pl.when(i < nk)
        def _():
            @pl.when(i + 1 < nk)
            def _():
                @pl.when(i >= 1)
                def _():
                    wait_scat(bo)

                issue_stage(i + 1, bo)

            wait_stage(b)

            @pl.when(c == 0)
            def _():
                pltpu.async_copy(rows[b], out0.at[idx[b]], sem_s[b], add=True)

            @pl.when(c == 1)
            def _():
                pltpu.async_copy(rows[b], out1.at[idx[b]], sem_s[b], add=True)

    issue_stage(0, 0)

    def pair(j, carry):
        step(2 * j, 0, 1)
        step(2 * j + 1, 1, 0)
        return carry

    lax.fori_loop(0, (n0 // 16 + 2) // 2, pair, 0)
    wait_scat(0)
    wait_scat(1)


def kernel(x, edge_index, edge_attr, W1, b1, W2, b2, Wa, ba, Wu, bu, gamma, beta):
    n, nin = x.shape
    e = edge_index.shape[1]
    ed = edge_attr.shape[1]
    oh = W2.shape[0]
    nh = Wa.shape[1]
    out = Wu.shape[1]
    f32 = jnp.float32

    src = edge_index[0]
    tgt = edge_index[1]

    b1r = b1.reshape(1, oh)
    b2r = b2.reshape(1, oh)
    bur = bu.reshape(1, out)
    gr = gamma.reshape(1, out)
    br = beta.reshape(1, out)
    wa16 = jnp.pad(Wa, ((0, 0), (0, 16 - nh)))
    ba16 = jnp.pad(ba.reshape(1, nh), ((0, 0), (0, 16 - nh)))
    w1b = W1.astype(jnp.bfloat16)
    w2b = W2.astype(jnp.bfloat16)
    eab16 = edge_attr.astype(jnp.bfloat16)
    sel = jnp.pad(b2.reshape(nh, out), ((0, 16 - nh), (0, 0)))

    t16, wc16, bc16 = pl.pallas_call(
        _prep_body,
        out_shape=[
            jax.ShapeDtypeStruct((n, 16), f32),
            jax.ShapeDtypeStruct((oh, 16), f32),
            jax.ShapeDtypeStruct((1, 16), f32),
        ],
    )(x, wa16, W2, b2r, ba16)
    wc16b = wc16.astype(jnp.bfloat16)
    t4 = t16[:, :nh].reshape(-1)

    mesh = plsc.VectorSubcoreMesh(core_axis_name="c", subcore_axis_name="s")
    gather_k = functools.partial(
        pl.kernel,
        out_type=[
            jax.ShapeDtypeStruct((e // 2, nin), f32),
            jax.ShapeDtypeStruct((e // 2, 16), f32),
        ],
        mesh=mesh,
        compiler_params=pltpu.CompilerParams(needs_layout_passes=False),
        scratch_types=[
            pltpu.VMEM((_GCH,), jnp.int32),
            pltpu.VMEM((_GCH,), jnp.int32),
            pltpu.VMEM((_GCH,), jnp.int32),
            pltpu.VMEM((_GCH,), jnp.int32),
            pltpu.VMEM((_GCH, nin), f32),
            pltpu.VMEM((_GCH, nin), f32),
            pltpu.VMEM((_GCH, 16), f32),
            pltpu.VMEM((_GCH, 16), f32),
            pltpu.VMEM((nh * n,), f32),
            pltpu.SemaphoreType.DMA,
            pltpu.SemaphoreType.DMA,
            pltpu.SemaphoreType.DMA,
            pltpu.SemaphoreType.DMA,
            pltpu.SemaphoreType.DMA,
            pltpu.SemaphoreType.DMA,
            pltpu.SemaphoreType.DMA,
        ],
    )(_sc_gather_body)

    eh = e // 2
    be = 2000
    be2 = 1600

    def run_gather(sl):
        return gather_k(x, t4, src[sl], tgt[sl])

    def run_pass1(xsrc_h, ea_h, ttgt_h):
        return pl.pallas_call(
            _pass1_body,
            grid=(eh // be,),
            in_specs=[
                pl.BlockSpec((be, nin), lambda i: (i, 0)),
                pl.BlockSpec((be, ed), lambda i: (i, 0)),
                pl.BlockSpec((be, 16), lambda i: (i, 0)),
                pl.BlockSpec((nin + ed, oh), lambda i: (0, 0)),
                pl.BlockSpec((1, oh), lambda i: (0, 0)),
                pl.BlockSpec((oh, 16), lambda i: (0, 0)),
                pl.BlockSpec((1, 16), lambda i: (0, 0)),
            ],
            out_specs=[
                pl.BlockSpec((be, 16), lambda i: (i, 0)),
                pl.BlockSpec((1, 16), lambda i: (0, 0)),
                pl.BlockSpec((1, 16), lambda i: (0, 0)),
            ],
            out_shape=[
                jax.ShapeDtypeStruct((eh, 16), f32),
                jax.ShapeDtypeStruct((1, 16), f32),
                jax.ShapeDtypeStruct((1, 16), f32),
            ],
            scratch_shapes=[
                pltpu.VMEM((1, 16), f32),
                pltpu.VMEM((1, 16), f32),
            ],
        )(xsrc_h, ea_h, ttgt_h, w1b, b1r, wc16b, bc16)

    def run_pass2(xsrc_h, ea_h, logits_h, m0, z0, m1, z1):
        return pl.pallas_call(
            _pass2_body,
            grid=(eh // be2,),
            in_specs=[
                pl.BlockSpec((be2, nin), lambda i: (i, 0)),
                pl.BlockSpec((be2, ed), lambda i: (i, 0)),
                pl.BlockSpec((be2, 16), lambda i: (i, 0)),
                pl.BlockSpec((1, 16), lambda i: (0, 0)),
                pl.BlockSpec((1, 16), lambda i: (0, 0)),
                pl.BlockSpec((1, 16), lambda i: (0, 0)),
                pl.BlockSpec((1, 16), lambda i: (0, 0)),
                pl.BlockSpec((nin + ed, oh), lambda i: (0, 0)),
                pl.BlockSpec((1, oh), lambda i: (0, 0)),
                pl.BlockSpec((oh, oh), lambda i: (0, 0)),
                pl.BlockSpec((16, out), lambda i: (0, 0)),
            ],
            out_specs=pl.BlockSpec((be2, out), lambda i: (i, 0)),
            out_shape=jax.ShapeDtypeStruct((eh, out), f32),
        )(xsrc_h, ea_h, logits_h, m0, z0, m1, z1, w1b, b1r, w2b, sel)

    zeros_hbm = jnp.zeros((_CH, out), f32)
    scatter_k = functools.partial(
        pl.kernel,
        out_type=[
            jax.ShapeDtypeStruct((10240, out), f32),
            jax.ShapeDtypeStruct((10240, out), f32),
        ],
        mesh=mesh,
        compiler_params=pltpu.CompilerParams(needs_layout_passes=False),
        scratch_types=[
            pltpu.VMEM((_CH,), jnp.int32),
            pltpu.VMEM((_CH,), jnp.int32),
            pltpu.VMEM((_CH, out), f32),
            pltpu.VMEM((_CH, out), f32),
            pltpu.SemaphoreType.DMA,
            pltpu.SemaphoreType.DMA,
            pltpu.SemaphoreType.DMA,
            pltpu.SemaphoreType.DMA,
            pltpu.SemaphoreType.DMA,
            pltpu.SemaphoreType.DMA,
        ],
    )(_sc_scatter_body)

    h0 = slice(0, eh)
    h1 = slice(eh, e)
    ea0, ea1 = eab16[h0], eab16[h1]
    xsrc0, ttgt0 = run_gather(h0)
    xsrc1, ttgt1 = run_gather(h1)
    logits0, m0, z0 = run_pass1(xsrc0, ea0, ttgt0)
    logits1, m1, z1 = run_pass1(xsrc1, ea1, ttgt1)
    weighted0 = run_pass2(xsrc0, ea0, logits0, m0, z0, m1, z1)
    agg0a, agg1a = scatter_k(weighted0, tgt[h0], zeros_hbm)
    weighted1 = run_pass2(xsrc1, ea1, logits1, m0, z0, m1, z1)
    agg0b, agg1b = scatter_k(weighted1, tgt[h1], zeros_hbm)

    bn = 2000
    return pl.pallas_call(
        _pass3_body,
        grid=(n // bn,),
        in_specs=[
            pl.BlockSpec((bn, nin), lambda i: (i, 0)),
            pl.BlockSpec((bn, out), lambda i: (i, 0)),
            pl.BlockSpec((bn, out), lambda i: (i, 0)),
            pl.BlockSpec((bn, out), lambda i: (i, 0)),
            pl.BlockSpec((bn, out), lambda i: (i, 0)),
            pl.BlockSpec((nin + out, out), lambda i: (0, 0)),
            pl.BlockSpec((1, out), lambda i: (0, 0)),
            pl.BlockSpec((1, out), lambda i: (0, 0)),
            pl.BlockSpec((1, out), lambda i: (0, 0)),
        ],
        out_specs=pl.BlockSpec((bn, out), lambda i: (i, 0)),
        out_shape=jax.ShapeDtypeStruct((n, out), f32),
    )(x, agg0a[:n], agg1a[:n], agg0b[:n], agg1b[:n], Wu, bur, gr, br)
